# baseline (speedup 1.0000x reference)
import jax
import jax.numpy as jnp
from jax.experimental import pallas as pl
from jax.experimental.pallas import tpu as pltpu

EPS = 1e-5
VMEM_LIMIT = 64 * 1024 * 1024


def _up2(a):
    H, W, C = a.shape
    a = jnp.broadcast_to(a[:, None], (H, 2, W, C)).reshape(2 * H, W, C)
    a = jnp.broadcast_to(a[:, :, None], (2 * H, W, 2, C)).reshape(2 * H, 2 * W, C)
    return a


def _build_colpat(cp_ref, up):
    H2, W2, C = up.shape
    z = jnp.zeros((1, W2, 3 * C), jnp.bfloat16)
    cp_ref[0:1] = z
    cp_ref[H2 + 1:H2 + 2] = z
    zc = jnp.zeros((H2, 1, C), jnp.bfloat16)
    cp_ref[1:H2 + 1, 0:1, 0:C] = zc
    cp_ref[1:H2 + 1, 1:W2, 0:C] = up[:, 0:W2 - 1, :]
    cp_ref[1:H2 + 1, :, C:2 * C] = up
    cp_ref[1:H2 + 1, 0:W2 - 1, 2 * C:3 * C] = up[:, 1:W2, :]
    cp_ref[1:H2 + 1, W2 - 1:W2, 2 * C:3 * C] = zc


def _conv3(cp_ref, w_ref, H2, W2, K):
    acc = None
    for dy in range(3):
        a = cp_ref[dy:dy + H2].reshape(H2 * W2, K)
        d = jnp.dot(a, w_ref[dy], preferred_element_type=jnp.float32)
        acc = d if acc is None else acc + d
    return acc


def _prep_kernel(x_ref, feat_ref, fcw_ref, fcb_ref, s1_ref, sh1_ref, aff2_ref):
    N, H, W, C = x_ref.shape
    M = N * H * W
    x = x_ref[...].reshape(M, C)
    aff = jnp.dot(feat_ref[...], fcw_ref[...],
                  preferred_element_type=jnp.float32) + fcb_ref[...]
    mean1 = jnp.sum(x, axis=0, keepdims=True) / M
    ex2 = jnp.sum(x * x, axis=0, keepdims=True) / M
    inv1 = jax.lax.rsqrt(ex2 - mean1 * mean1 + EPS)
    wv1 = aff[:, 0 * C:1 * C]
    bv1 = aff[:, 1 * C:2 * C]
    s1_ref[...] = (wv1 * inv1).reshape(N, 1, C)
    sh1_ref[...] = (bv1 - wv1 * mean1 * inv1).reshape(N, 1, C)
    aff2_ref[...] = aff[:, 2 * C:4 * C].reshape(N, 1, 2 * C)


def _conv1_kernel(x_ref, s1_ref, sh1_ref, w1_ref, b1_ref, t_ref, part_ref,
                  cp_ref):
    _, H, W, C = x_ref.shape
    H2, W2 = 2 * H, 2 * W
    x = x_ref[0].reshape(H * W, C)
    h = jnp.maximum(s1_ref[0] * x + sh1_ref[0], 0.0).astype(jnp.bfloat16)
    up = _up2(h.reshape(H, W, C))
    _build_colpat(cp_ref, up)
    conv = _conv3(cp_ref, w1_ref, H2, W2, 3 * C) + b1_ref[...]
    part_ref[0, 0:1, :] = jnp.sum(conv, axis=0, keepdims=True)
    part_ref[0, 1:2, :] = jnp.sum(conv * conv, axis=0, keepdims=True)
    t_ref[0] = conv.astype(jnp.bfloat16).reshape(H2, W2, C)


def _conv2_kernel(t_ref, part_ref, aff2_ref, x_ref, w3_ref, b3_ref, w2_ref,
                  b2_ref, out_ref, cp_ref):
    _, H2, W2, C = t_ref.shape
    H, W = H2 // 2, W2 // 2
    N = part_ref.shape[0]
    Cout = w3_ref.shape[1]
    M2 = N * H2 * W2
    mean2 = jnp.sum(part_ref[:, 0, :], axis=0, keepdims=True) / M2
    ex2 = jnp.sum(part_ref[:, 1, :], axis=0, keepdims=True) / M2
    inv2 = jax.lax.rsqrt(ex2 - mean2 * mean2 + EPS)
    wv2 = aff2_ref[0, :, 0:C]
    bv2 = aff2_ref[0, :, C:2 * C]
    sc = wv2 * inv2
    sh = bv2 - wv2 * mean2 * inv2
    t = t_ref[0].reshape(H2 * W2, C).astype(jnp.float32)
    z = jnp.maximum(sc * t + sh, 0.0).astype(jnp.bfloat16)
    _build_colpat(cp_ref, z.reshape(H2, W2, C))
    conv = _conv3(cp_ref, w2_ref, H2, W2, 3 * C) + b2_ref[...]
    xs = x_ref[0].reshape(H * W, C).astype(jnp.bfloat16)
    skip = jnp.dot(xs, w3_ref[...],
                   preferred_element_type=jnp.float32) + b3_ref[...]
    sku = _up2(skip.reshape(H, W, Cout)).reshape(H2 * W2, Cout)
    out_ref[0] = (conv + sku).reshape(H2, W2, Cout)


def kernel(x, feat, w1, b1, w2, b2, w3, b3, fcw1_w, fcw1_b, fcb1_w, fcb1_b,
           fcw2_w, fcw2_b, fcb2_w, fcb2_b):
    N, Cin, H, W = x.shape
    Cout = w2.shape[0]
    H2, W2 = 2 * H, 2 * W
    f32, bf16 = jnp.float32, jnp.bfloat16

    x_lo = jnp.transpose(x, (0, 2, 3, 1)).astype(f32)
    feat = feat.astype(f32)
    w1r = jnp.transpose(w1, (2, 3, 1, 0)).reshape(3, 3 * Cin, Cin).astype(bf16)
    w2r = jnp.transpose(w2, (2, 3, 1, 0)).reshape(3, 3 * Cin, Cout).astype(bf16)
    w3m = jnp.transpose(w3[:, :, 0, 0], (1, 0)).astype(bf16)
    b1m = b1.reshape(1, Cin).astype(f32)
    b2m = b2.reshape(1, Cout).astype(f32)
    b3m = b3.reshape(1, Cout).astype(f32)
    fc_w = jnp.concatenate([fcw1_w, fcb1_w, fcw2_w, fcb2_w], axis=1).astype(f32)
    fc_b = jnp.concatenate([fcw1_b, fcb1_b, fcw2_b, fcb2_b]).reshape(1, 4 * Cin)

    vmem = pl.BlockSpec(memory_space=pltpu.MemorySpace.VMEM)

    s1, sh1, aff2 = pl.pallas_call(
        _prep_kernel,
        out_shape=(jax.ShapeDtypeStruct((N, 1, Cin), f32),
                   jax.ShapeDtypeStruct((N, 1, Cin), f32),
                   jax.ShapeDtypeStruct((N, 1, 2 * Cin), f32)),
        in_specs=[vmem] * 4,
        out_specs=(vmem, vmem, vmem),
        compiler_params=pltpu.CompilerParams(vmem_limit_bytes=VMEM_LIMIT),
    )(x_lo, feat, fc_w, fc_b)

    t, part = pl.pallas_call(
        _conv1_kernel,
        out_shape=(jax.ShapeDtypeStruct((N, H2, W2, Cin), bf16),
                   jax.ShapeDtypeStruct((N, 2, Cin), f32)),
        grid=(N,),
        in_specs=[pl.BlockSpec((1, H, W, Cin), lambda n: (n, 0, 0, 0)),
                  pl.BlockSpec((1, 1, Cin), lambda n: (n, 0, 0)),
                  pl.BlockSpec((1, 1, Cin), lambda n: (n, 0, 0)),
                  pl.BlockSpec((3, 3 * Cin, Cin), lambda n: (0, 0, 0)),
                  pl.BlockSpec((1, Cin), lambda n: (0, 0))],
        out_specs=(pl.BlockSpec((1, H2, W2, Cin), lambda n: (n, 0, 0, 0)),
                   pl.BlockSpec((1, 2, Cin), lambda n: (n, 0, 0))),
        scratch_shapes=[pltpu.VMEM((H2 + 2, W2, 3 * Cin), bf16)],
        compiler_params=pltpu.CompilerParams(
            dimension_semantics=("parallel",), vmem_limit_bytes=VMEM_LIMIT),
    )(x_lo, s1, sh1, w1r, b1m)

    out_nhwc = pl.pallas_call(
        _conv2_kernel,
        out_shape=jax.ShapeDtypeStruct((N, H2, W2, Cout), f32),
        grid=(N,),
        in_specs=[pl.BlockSpec((1, H2, W2, Cin), lambda n: (n, 0, 0, 0)),
                  pl.BlockSpec((N, 2, Cin), lambda n: (0, 0, 0)),
                  pl.BlockSpec((1, 1, 2 * Cin), lambda n: (n, 0, 0)),
                  pl.BlockSpec((1, H, W, Cin), lambda n: (n, 0, 0, 0)),
                  pl.BlockSpec((Cin, Cout), lambda n: (0, 0)),
                  pl.BlockSpec((1, Cout), lambda n: (0, 0)),
                  pl.BlockSpec((3, 3 * Cin, Cout), lambda n: (0, 0, 0)),
                  pl.BlockSpec((1, Cout), lambda n: (0, 0))],
        out_specs=pl.BlockSpec((1, H2, W2, Cout), lambda n: (n, 0, 0, 0)),
        scratch_shapes=[pltpu.VMEM((H2 + 2, W2, 3 * Cin), bf16)],
        compiler_params=pltpu.CompilerParams(
            dimension_semantics=("parallel",), vmem_limit_bytes=VMEM_LIMIT),
    )(t, part, aff2, x_lo, w3m, b3m, w2r, b2m)

    return jnp.transpose(out_nhwc, (0, 3, 1, 2))


# speedup vs baseline: 1.2507x; 1.2507x over previous
import jax
import jax.numpy as jnp
from jax.experimental import pallas as pl
from jax.experimental.pallas import tpu as pltpu

EPS = 1e-5
VMEM_LIMIT = 32 * 1024 * 1024


def _up2(a):
    H, W, C = a.shape
    a = jnp.broadcast_to(a[:, None], (H, 2, W, C)).reshape(2 * H, W, C)
    a = jnp.broadcast_to(a[:, :, None], (2 * H, W, 2, C)).reshape(2 * H, 2 * W, C)
    return a


def _build_colpat(cp_ref, up):
    H2, W2, C = up.shape
    z = jnp.zeros((1, W2, 3 * C), jnp.float32)
    cp_ref[0:1] = z
    cp_ref[H2 + 1:H2 + 2] = z
    zc = jnp.zeros((H2, 1, C), jnp.float32)
    cp_ref[1:H2 + 1, 0:1, 0:C] = zc
    cp_ref[1:H2 + 1, 1:W2, 0:C] = up[:, 0:W2 - 1, :]
    cp_ref[1:H2 + 1, :, C:2 * C] = up
    cp_ref[1:H2 + 1, 0:W2 - 1, 2 * C:3 * C] = up[:, 1:W2, :]
    cp_ref[1:H2 + 1, W2 - 1:W2, 2 * C:3 * C] = zc


def _conv3(cp_ref, w_ref, H2, W2, K):
    flat = cp_ref[...].reshape((H2 + 2) * W2, K)
    acc = None
    for dy in range(3):
        a = flat[dy * W2:dy * W2 + H2 * W2]
        d = jnp.dot(a, w_ref[dy], preferred_element_type=jnp.float32)
        acc = d if acc is None else acc + d
    return acc


def _prep_kernel(x_ref, feat_ref, fcw_ref, fcb_ref, s1_ref, sh1_ref, aff2_ref):
    N, H, W, C = x_ref.shape
    M = N * H * W
    x = x_ref[...].reshape(M, C)
    aff = jnp.dot(feat_ref[...], fcw_ref[...],
                  preferred_element_type=jnp.float32) + fcb_ref[...]
    mean1 = jnp.sum(x, axis=0, keepdims=True) / M
    ex2 = jnp.sum(x * x, axis=0, keepdims=True) / M
    inv1 = jax.lax.rsqrt(ex2 - mean1 * mean1 + EPS)
    wv1 = aff[:, 0 * C:1 * C]
    bv1 = aff[:, 1 * C:2 * C]
    s1_ref[...] = (wv1 * inv1).reshape(N, 1, C)
    sh1_ref[...] = (bv1 - wv1 * mean1 * inv1).reshape(N, 1, C)
    aff2_ref[...] = aff[:, 2 * C:4 * C].reshape(N, 1, 2 * C)


def _conv1_kernel(x_ref, s1_ref, sh1_ref, w1_ref, b1_ref, t_ref, part_ref,
                  cp_ref):
    _, H, W, C = x_ref.shape
    H2, W2 = 2 * H, 2 * W
    x = x_ref[0].reshape(H * W, C)
    h = jnp.maximum(s1_ref[0] * x + sh1_ref[0], 0.0)
    up = _up2(h.reshape(H, W, C))
    _build_colpat(cp_ref, up)
    conv = _conv3(cp_ref, w1_ref, H2, W2, 3 * C) + b1_ref[...]
    part_ref[0, 0:1, :] = jnp.sum(conv, axis=0, keepdims=True)
    part_ref[0, 1:2, :] = jnp.sum(conv * conv, axis=0, keepdims=True)
    t_ref[0] = conv.astype(jnp.bfloat16).reshape(H2, W2, C)


def _conv2_kernel(t_ref, part_ref, aff2_ref, x_ref, w3_ref, b3_ref, w2_ref,
                  b2_ref, out_ref, cp_ref):
    _, H2, W2, C = t_ref.shape
    H, W = H2 // 2, W2 // 2
    N = part_ref.shape[0]
    Cout = w3_ref.shape[1]
    M2 = N * H2 * W2
    mean2 = jnp.sum(part_ref[:, 0, :], axis=0, keepdims=True) / M2
    ex2 = jnp.sum(part_ref[:, 1, :], axis=0, keepdims=True) / M2
    inv2 = jax.lax.rsqrt(ex2 - mean2 * mean2 + EPS)
    wv2 = aff2_ref[0, :, 0:C]
    bv2 = aff2_ref[0, :, C:2 * C]
    sc = wv2 * inv2
    sh = bv2 - wv2 * mean2 * inv2
    t = t_ref[0].reshape(H2 * W2, C).astype(jnp.float32)
    z = jnp.maximum(sc * t + sh, 0.0)
    _build_colpat(cp_ref, z.reshape(H2, W2, C))
    conv = _conv3(cp_ref, w2_ref, H2, W2, 3 * C) + b2_ref[...]
    xs = x_ref[0].reshape(H * W, C)
    skip = jnp.dot(xs, w3_ref[...],
                   preferred_element_type=jnp.float32) + b3_ref[...]
    sku = _up2(skip.reshape(H, W, Cout)).reshape(H2 * W2, Cout)
    out_ref[0] = (conv + sku).reshape(H2, W2, Cout)


def kernel(x, feat, w1, b1, w2, b2, w3, b3, fcw1_w, fcw1_b, fcb1_w, fcb1_b,
           fcw2_w, fcw2_b, fcb2_w, fcb2_b):
    N, Cin, H, W = x.shape
    Cout = w2.shape[0]
    H2, W2 = 2 * H, 2 * W
    f32, bf16 = jnp.float32, jnp.bfloat16

    x_lo = jnp.transpose(x, (0, 2, 3, 1)).astype(f32)
    feat = feat.astype(f32)
    w1r = jnp.transpose(w1, (2, 3, 1, 0)).reshape(3, 3 * Cin, Cin).astype(f32)
    w2r = jnp.transpose(w2, (2, 3, 1, 0)).reshape(3, 3 * Cin, Cout).astype(f32)
    w3m = jnp.transpose(w3[:, :, 0, 0], (1, 0)).astype(f32)
    b1m = b1.reshape(1, Cin).astype(f32)
    b2m = b2.reshape(1, Cout).astype(f32)
    b3m = b3.reshape(1, Cout).astype(f32)
    fc_w = jnp.concatenate([fcw1_w, fcb1_w, fcw2_w, fcb2_w], axis=1).astype(f32)
    fc_b = jnp.concatenate([fcw1_b, fcb1_b, fcw2_b, fcb2_b]).reshape(1, 4 * Cin)

    vmem = pl.BlockSpec(memory_space=pltpu.MemorySpace.VMEM)

    s1, sh1, aff2 = pl.pallas_call(
        _prep_kernel,
        out_shape=(jax.ShapeDtypeStruct((N, 1, Cin), f32),
                   jax.ShapeDtypeStruct((N, 1, Cin), f32),
                   jax.ShapeDtypeStruct((N, 1, 2 * Cin), f32)),
        in_specs=[vmem] * 4,
        out_specs=(vmem, vmem, vmem),
        compiler_params=pltpu.CompilerParams(vmem_limit_bytes=VMEM_LIMIT),
    )(x_lo, feat, fc_w, fc_b)

    t, part = pl.pallas_call(
        _conv1_kernel,
        out_shape=(jax.ShapeDtypeStruct((N, H2, W2, Cin), bf16),
                   jax.ShapeDtypeStruct((N, 2, Cin), f32)),
        grid=(N,),
        in_specs=[pl.BlockSpec((1, H, W, Cin), lambda n: (n, 0, 0, 0)),
                  pl.BlockSpec((1, 1, Cin), lambda n: (n, 0, 0)),
                  pl.BlockSpec((1, 1, Cin), lambda n: (n, 0, 0)),
                  pl.BlockSpec((3, 3 * Cin, Cin), lambda n: (0, 0, 0)),
                  pl.BlockSpec((1, Cin), lambda n: (0, 0))],
        out_specs=(pl.BlockSpec((1, H2, W2, Cin), lambda n: (n, 0, 0, 0)),
                   pl.BlockSpec((1, 2, Cin), lambda n: (n, 0, 0))),
        scratch_shapes=[pltpu.VMEM((H2 + 2, W2, 3 * Cin), f32)],
        compiler_params=pltpu.CompilerParams(
            dimension_semantics=("parallel",), vmem_limit_bytes=VMEM_LIMIT),
    )(x_lo, s1, sh1, w1r, b1m)

    out_nhwc = pl.pallas_call(
        _conv2_kernel,
        out_shape=jax.ShapeDtypeStruct((N, H2, W2, Cout), f32),
        grid=(N,),
        in_specs=[pl.BlockSpec((1, H2, W2, Cin), lambda n: (n, 0, 0, 0)),
                  pl.BlockSpec((N, 2, Cin), lambda n: (0, 0, 0)),
                  pl.BlockSpec((1, 1, 2 * Cin), lambda n: (n, 0, 0)),
                  pl.BlockSpec((1, H, W, Cin), lambda n: (n, 0, 0, 0)),
                  pl.BlockSpec((Cin, Cout), lambda n: (0, 0)),
                  pl.BlockSpec((1, Cout), lambda n: (0, 0)),
                  pl.BlockSpec((3, 3 * Cin, Cout), lambda n: (0, 0, 0)),
                  pl.BlockSpec((1, Cout), lambda n: (0, 0))],
        out_specs=pl.BlockSpec((1, H2, W2, Cout), lambda n: (n, 0, 0, 0)),
        scratch_shapes=[pltpu.VMEM((H2 + 2, W2, 3 * Cin), f32)],
        compiler_params=pltpu.CompilerParams(
            dimension_semantics=("parallel",), vmem_limit_bytes=VMEM_LIMIT),
    )(t, part, aff2, x_lo, w3m, b3m, w2r, b2m)

    return jnp.transpose(out_nhwc, (0, 3, 1, 2))
